# speedup vs baseline: 16.0601x; 16.0601x over previous
import jax
import jax.numpy as jnp
from jax.experimental import pallas as pl

N_QUANT = 256
B = 16
T = 8192
T_BLK = 512


def _onehot_body(idx_ref, out_ref):
    idx = idx_ref[...]
    q = jax.lax.broadcasted_iota(jnp.int32, (B, N_QUANT, T_BLK), 1)
    out_ref[...] = (q == idx[:, None, :]).astype(jnp.float32)


def kernel(quant_onehot, in_snd_slice):
    del quant_onehot
    idx = in_snd_slice.astype(jnp.int32)
    return pl.pallas_call(
        _onehot_body,
        grid=(T // T_BLK,),
        in_specs=[pl.BlockSpec((B, T_BLK), lambda i: (0, i))],
        out_specs=pl.BlockSpec((B, N_QUANT, T_BLK), lambda i: (0, 0, i)),
        out_shape=jax.ShapeDtypeStruct((B, N_QUANT, T), jnp.float32),
    )(idx)
